# baseline (speedup 1.0000x reference)
import jax
import jax.numpy as jnp
from jax.experimental import pallas as pl
from jax.experimental.pallas import tpu as pltpu


def _copy(u_ref, ou_ref):
    ou_ref[...] = u_ref[...]


def kernel(unary, binary, index1, index2):
    u2 = unary.reshape(3125, 128)
    out_u = pl.pallas_call(
        _copy,
        out_shape=jax.ShapeDtypeStruct((3125, 128), jnp.float32),
    )(u2)
    return out_u.reshape(unary.shape), binary
